# Initial kernel scaffold; baseline (speedup 1.0000x reference)
import jax
import jax.numpy as jnp
from jax.experimental import pallas as pl


def kernel(coords, edge_index, embedding):
    raise NotImplementedError("write your pallas kernel here")


# speedup vs baseline: 12.3133x; 12.3133x over previous
import functools

import jax
import jax.numpy as jnp
from jax import lax
from jax.experimental import pallas as pl
from jax.experimental.pallas import tpu as pltpu
from jax.experimental.pallas import tpu_sc as plsc

N_HEADS = 8
D_K = 16
N_BUCKETS = 32
LANES = 16
NUM_WORKERS = 32


def _sq_dist_sc(coords_flat, edges_flat, *, B, N, E):
    total = B * E
    epw = total // NUM_WORKERS
    mesh = plsc.VectorSubcoreMesh(core_axis_name="c", subcore_axis_name="s")

    @functools.partial(
        pl.kernel,
        mesh=mesh,
        out_type=jax.ShapeDtypeStruct((total,), jnp.float32),
        scratch_types=[
            pltpu.VMEM((2 * N,), jnp.float32),
            pltpu.VMEM((2 * epw,), jnp.int32),
            pltpu.VMEM((epw,), jnp.float32),
        ],
        compiler_params=pltpu.CompilerParams(needs_layout_passes=False),
    )
    def k(coords_hbm, edges_hbm, out_hbm, coords_v, edges_v, d2_v):
        wid = lax.axis_index("s") * 2 + lax.axis_index("c")
        base = wid * epw
        b = base // E
        pltpu.sync_copy(coords_hbm.at[pl.ds(b * 2 * N, 2 * N)], coords_v)
        pltpu.sync_copy(edges_hbm.at[pl.ds(base * 2, 2 * epw)], edges_v)
        lanes = lax.iota(jnp.int32, LANES)

        def body(i, carry):
            p = i * LANES
            ei = 2 * (p + lanes)
            src = plsc.load_gather(edges_v, [ei])
            dst = plsc.load_gather(edges_v, [ei + 1])
            sx = plsc.load_gather(coords_v, [2 * src])
            sy = plsc.load_gather(coords_v, [2 * src + 1])
            tx = plsc.load_gather(coords_v, [2 * dst])
            ty = plsc.load_gather(coords_v, [2 * dst + 1])
            dx = sx - tx
            dy = sy - ty
            d2_v[pl.ds(p, LANES)] = dx * dx + dy * dy
            return carry

        lax.fori_loop(0, epw // LANES, body, 0)
        pltpu.sync_copy(d2_v, out_hbm.at[pl.ds(base, epw)])

    return k(coords_flat, edges_flat)


def _lookup_tc(d2, embedding):
    total = d2.shape[0]
    blk = 2560
    d_model = embedding.shape[1]

    def body(d2_ref, emb_ref, out_ref):
        d2v = d2_ref[...]
        dist = jnp.sqrt(d2v)
        bkt = jnp.clip((dist * N_BUCKETS).astype(jnp.int32), 0, N_BUCKETS - 1)
        bktb = jnp.broadcast_to(bkt, (N_BUCKETS, blk))
        iot = lax.broadcasted_iota(jnp.int32, (N_BUCKETS, blk), 0)
        onehot_t = (bktb == iot).astype(jnp.float32)
        out_ref[...] = lax.dot_general(
            onehot_t, emb_ref[...],
            dimension_numbers=(((0,), (0,)), ((), ())),
            preferred_element_type=jnp.float32)

    return pl.pallas_call(
        body,
        grid=(total // blk,),
        in_specs=[
            pl.BlockSpec((1, blk), lambda i: (0, i)),
            pl.BlockSpec((N_BUCKETS, d_model), lambda i: (0, 0)),
        ],
        out_specs=pl.BlockSpec((blk, d_model), lambda i: (i, 0)),
        out_shape=jax.ShapeDtypeStruct((total, d_model), jnp.float32),
    )(d2.reshape(1, total), embedding)


def kernel(coords, edge_index, embedding):
    B, N, _ = coords.shape
    _, E, _ = edge_index.shape
    coords_flat = coords.reshape(-1)
    edges_flat = edge_index.astype(jnp.int32).reshape(-1)
    d2 = _sq_dist_sc(coords_flat, edges_flat, B=B, N=N, E=E)
    out = _lookup_tc(d2, embedding)
    return out.reshape(B, E, N_HEADS, D_K)


# speedup vs baseline: 86.9961x; 7.0652x over previous
import functools

import jax
import jax.numpy as jnp
from jax import lax
from jax.experimental import pallas as pl
from jax.experimental.pallas import tpu as pltpu
from jax.experimental.pallas import tpu_sc as plsc

N_HEADS = 8
D_K = 16
N_BUCKETS = 32
LANES = 16
NUM_WORKERS = 32
ETILE = 128


def _sq_dist_sc(coords_flat, ei_lin, *, B, N, E):
    total = B * E
    tpb = E // ETILE
    wpb = NUM_WORKERS // B
    base_t = tpb // wpb
    rem = tpb - base_t * wpb
    mesh = plsc.VectorSubcoreMesh(core_axis_name="c", subcore_axis_name="s")

    @functools.partial(
        pl.kernel,
        mesh=mesh,
        out_type=jax.ShapeDtypeStruct((total,), jnp.float32),
        scratch_types=[
            pltpu.VMEM((2 * N,), jnp.float32),
            pltpu.VMEM((base_t * 2 * ETILE,), jnp.int32),
            pltpu.VMEM((base_t * ETILE,), jnp.float32),
            pltpu.VMEM((2 * ETILE,), jnp.int32),
            pltpu.VMEM((ETILE,), jnp.float32),
        ],
        compiler_params=pltpu.CompilerParams(needs_layout_passes=False),
    )
    def k(coords_hbm, edges_hbm, out_hbm, coords_v, ebuf, dbuf, eext, dext):
        wid = lax.axis_index("s") * 2 + lax.axis_index("c")
        b = wid // wpb
        j = wid % wpb
        t0 = b * tpb + j * base_t
        pltpu.sync_copy(coords_hbm.at[pl.ds(b * 2 * N, 2 * N)], coords_v)
        pltpu.sync_copy(edges_hbm.at[pl.ds(t0 * 2 * ETILE, base_t * 2 * ETILE)],
                        ebuf)

        def do16(src, dst):
            sx = plsc.load_gather(coords_v, [2 * src])
            sy = plsc.load_gather(coords_v, [2 * src + 1])
            tx = plsc.load_gather(coords_v, [2 * dst])
            ty = plsc.load_gather(coords_v, [2 * dst + 1])
            dx = sx - tx
            dy = sy - ty
            return dx * dx + dy * dy

        def tile_body(t, carry):
            eb = t * 2 * ETILE
            db = t * ETILE
            for q in range(ETILE // LANES):
                src = ebuf[pl.ds(eb + q * LANES, LANES)]
                dst = ebuf[pl.ds(eb + ETILE + q * LANES, LANES)]
                dbuf[pl.ds(db + q * LANES, LANES)] = do16(src, dst)
            return carry

        lax.fori_loop(0, base_t, tile_body, 0)
        pltpu.sync_copy(dbuf, out_hbm.at[pl.ds(t0 * ETILE, base_t * ETILE)])

        @pl.when(j < rem)
        def _():
            te = b * tpb + wpb * base_t + j
            pltpu.sync_copy(edges_hbm.at[pl.ds(te * 2 * ETILE, 2 * ETILE)],
                            eext)
            for q in range(ETILE // LANES):
                src = eext[pl.ds(q * LANES, LANES)]
                dst = eext[pl.ds(ETILE + q * LANES, LANES)]
                dext[pl.ds(q * LANES, LANES)] = do16(src, dst)
            pltpu.sync_copy(dext, out_hbm.at[pl.ds(te * ETILE, ETILE)])

    return k(coords_flat, ei_lin)


def _lookup_tc(d2, embedding):
    B, _, E = d2.shape
    eblk = 2560
    d_model = embedding.shape[1]

    def body(d2_ref, emb_ref, out_ref):
        d2v = d2_ref[0]
        dist = jnp.sqrt(d2v)
        bkt = jnp.clip((dist * N_BUCKETS).astype(jnp.int32), 0, N_BUCKETS - 1)
        bktb = jnp.broadcast_to(bkt, (N_BUCKETS, eblk))
        iot = lax.broadcasted_iota(jnp.int32, (N_BUCKETS, eblk), 0)
        onehot_t = (bktb == iot).astype(jnp.float32)
        mat = lax.dot_general(
            emb_ref[...], onehot_t,
            dimension_numbers=(((0,), (0,)), ((), ())),
            preferred_element_type=jnp.float32)
        for h in range(N_HEADS):
            out_ref[0, h] = mat[h * D_K:(h + 1) * D_K, :]

    return pl.pallas_call(
        body,
        grid=(B, E // eblk),
        in_specs=[
            pl.BlockSpec((1, 1, eblk), lambda b, i: (b, 0, i)),
            pl.BlockSpec((N_BUCKETS, d_model), lambda b, i: (0, 0)),
        ],
        out_specs=pl.BlockSpec((1, N_HEADS, D_K, eblk),
                               lambda b, i: (b, 0, 0, i)),
        out_shape=jax.ShapeDtypeStruct((B, N_HEADS, D_K, E), jnp.float32),
    )(d2, embedding)


def kernel(coords, edge_index, embedding):
    B, N, _ = coords.shape
    _, E, _ = edge_index.shape
    coords_flat = coords.reshape(-1)
    ntiles = E // ETILE
    ei_lin = (edge_index.astype(jnp.int32)
              .reshape(B, ntiles, ETILE, 2)
              .transpose(0, 1, 3, 2)
              .reshape(-1))
    d2 = _sq_dist_sc(coords_flat, ei_lin, B=B, N=N, E=E)
    out = _lookup_tc(d2.reshape(B, 1, E), embedding)
    return out.transpose(0, 3, 1, 2)


# speedup vs baseline: 123.2837x; 1.4171x over previous
import functools

import jax
import jax.numpy as jnp
from jax import lax
from jax.experimental import pallas as pl
from jax.experimental.pallas import tpu as pltpu
from jax.experimental.pallas import tpu_sc as plsc

N_HEADS = 8
D_K = 16
N_BUCKETS = 32
LANES = 16
NUM_WORKERS = 32
ETILE = 128


def _sq_dist_sc(coords_flat, ei_lin, *, B, N, E):
    total = B * E
    tpb = E // ETILE
    wpb = NUM_WORKERS // B
    base_t = tpb // wpb
    rem = tpb - base_t * wpb
    mesh = plsc.VectorSubcoreMesh(core_axis_name="c", subcore_axis_name="s")

    @functools.partial(
        pl.kernel,
        mesh=mesh,
        out_type=jax.ShapeDtypeStruct((total,), jnp.float32),
        scratch_types=[
            pltpu.VMEM((2 * N,), jnp.float32),
            pltpu.VMEM((base_t * 2 * ETILE,), jnp.int32),
            pltpu.VMEM((base_t * ETILE,), jnp.float32),
            pltpu.VMEM((2 * ETILE,), jnp.int32),
            pltpu.VMEM((ETILE,), jnp.float32),
        ],
        compiler_params=pltpu.CompilerParams(needs_layout_passes=False),
    )
    def k(coords_hbm, edges_hbm, out_hbm, coords_v, ebuf, dbuf, eext, dext):
        wid = lax.axis_index("s") * 2 + lax.axis_index("c")
        b = wid // wpb
        j = wid % wpb
        t0 = b * tpb + j * base_t
        pltpu.sync_copy(coords_hbm.at[pl.ds(b * 2 * N, 2 * N)], coords_v)
        pltpu.sync_copy(edges_hbm.at[pl.ds(t0 * 2 * ETILE, base_t * 2 * ETILE)],
                        ebuf)

        def do16(src, dst):
            sx = plsc.load_gather(coords_v, [2 * src])
            sy = plsc.load_gather(coords_v, [2 * src + 1])
            tx = plsc.load_gather(coords_v, [2 * dst])
            ty = plsc.load_gather(coords_v, [2 * dst + 1])
            dx = sx - tx
            dy = sy - ty
            return dx * dx + dy * dy

        def tile_body(t, carry):
            eb = t * 2 * ETILE
            db = t * ETILE
            for q in range(ETILE // LANES):
                src = ebuf[pl.ds(eb + q * LANES, LANES)]
                dst = ebuf[pl.ds(eb + ETILE + q * LANES, LANES)]
                dbuf[pl.ds(db + q * LANES, LANES)] = do16(src, dst)
            return carry

        lax.fori_loop(0, base_t, tile_body, 0)
        pltpu.sync_copy(dbuf, out_hbm.at[pl.ds(t0 * ETILE, base_t * ETILE)])

        @pl.when(j < rem)
        def _():
            te = b * tpb + wpb * base_t + j
            pltpu.sync_copy(edges_hbm.at[pl.ds(te * 2 * ETILE, 2 * ETILE)],
                            eext)
            for q in range(ETILE // LANES):
                src = eext[pl.ds(q * LANES, LANES)]
                dst = eext[pl.ds(ETILE + q * LANES, LANES)]
                dext[pl.ds(q * LANES, LANES)] = do16(src, dst)
            pltpu.sync_copy(dext, out_hbm.at[pl.ds(te * ETILE, ETILE)])

    return k(coords_flat, ei_lin)


def _lookup_tc(d2, embedding):
    B, _, E = d2.shape
    eblk = 6400
    d_model = embedding.shape[1]

    def body(d2_ref, emb_ref, out_ref):
        d2v = d2_ref[0]
        dist = jnp.sqrt(d2v)
        bkt = jnp.clip((dist * N_BUCKETS).astype(jnp.int32), 0, N_BUCKETS - 1)
        bktb = jnp.broadcast_to(bkt, (N_BUCKETS, eblk))
        iot = lax.broadcasted_iota(jnp.int32, (N_BUCKETS, eblk), 0)
        onehot_t = (bktb == iot).astype(jnp.float32)
        mat = lax.dot_general(
            emb_ref[...], onehot_t,
            dimension_numbers=(((0,), (0,)), ((), ())),
            preferred_element_type=jnp.float32)
        for h in range(N_HEADS):
            out_ref[0, h] = mat[h * D_K:(h + 1) * D_K, :]

    return pl.pallas_call(
        body,
        grid=(B, E // eblk),
        in_specs=[
            pl.BlockSpec((1, 1, eblk), lambda b, i: (b, 0, i)),
            pl.BlockSpec((N_BUCKETS, d_model), lambda b, i: (0, 0)),
        ],
        out_specs=pl.BlockSpec((1, N_HEADS, D_K, eblk),
                               lambda b, i: (b, 0, 0, i)),
        out_shape=jax.ShapeDtypeStruct((B, N_HEADS, D_K, E), jnp.float32),
    )(d2, embedding)


def kernel(coords, edge_index, embedding):
    B, N, _ = coords.shape
    _, E, _ = edge_index.shape
    coords_flat = coords.reshape(-1)
    ntiles = E // ETILE
    ei_lin = (edge_index.astype(jnp.int32)
              .reshape(B, ntiles, ETILE, 2)
              .transpose(0, 1, 3, 2)
              .reshape(-1))
    d2 = _sq_dist_sc(coords_flat, ei_lin, B=B, N=N, E=E)
    out = _lookup_tc(d2.reshape(B, 1, E), embedding)
    return out.transpose(0, 3, 1, 2)


# speedup vs baseline: 139.5430x; 1.1319x over previous
import functools

import jax
import jax.numpy as jnp
from jax import lax
from jax.experimental import pallas as pl
from jax.experimental.pallas import tpu as pltpu
from jax.experimental.pallas import tpu_sc as plsc

N_HEADS = 8
D_K = 16
N_BUCKETS = 32
LANES = 16
NUM_WORKERS = 32
ETILE = 128


def _sq_dist_sc(coords_flat, ei_lin, *, B, N, E):
    total = B * E
    tpb = E // ETILE
    wpb = NUM_WORKERS // B
    base_t = tpb // wpb
    rem = tpb - base_t * wpb
    mesh = plsc.VectorSubcoreMesh(core_axis_name="c", subcore_axis_name="s")

    @functools.partial(
        pl.kernel,
        mesh=mesh,
        out_type=jax.ShapeDtypeStruct((total,), jnp.float32),
        scratch_types=[
            pltpu.VMEM((2 * N,), jnp.float32),
            pltpu.VMEM((base_t * 2 * ETILE,), jnp.int32),
            pltpu.VMEM((base_t * ETILE,), jnp.float32),
            pltpu.VMEM((2 * ETILE,), jnp.int32),
            pltpu.VMEM((ETILE,), jnp.float32),
        ],
        compiler_params=pltpu.CompilerParams(needs_layout_passes=False),
    )
    def k(coords_hbm, edges_hbm, out_hbm, coords_v, ebuf, dbuf, eext, dext):
        wid = lax.axis_index("s") * 2 + lax.axis_index("c")
        b = wid // wpb
        j = wid % wpb
        t0 = b * tpb + j * base_t
        pltpu.sync_copy(coords_hbm.at[pl.ds(b * 2 * N, 2 * N)], coords_v)
        pltpu.sync_copy(edges_hbm.at[pl.ds(t0 * 2 * ETILE, base_t * 2 * ETILE)],
                        ebuf)

        def do16(src, dst):
            sx = plsc.load_gather(coords_v, [2 * src])
            sy = plsc.load_gather(coords_v, [2 * src + 1])
            tx = plsc.load_gather(coords_v, [2 * dst])
            ty = plsc.load_gather(coords_v, [2 * dst + 1])
            dx = sx - tx
            dy = sy - ty
            return dx * dx + dy * dy

        def tile_body(t, carry):
            eb = t * 2 * ETILE
            db = t * ETILE
            for q in range(ETILE // LANES):
                src = ebuf[pl.ds(eb + q * LANES, LANES)]
                dst = ebuf[pl.ds(eb + ETILE + q * LANES, LANES)]
                dbuf[pl.ds(db + q * LANES, LANES)] = do16(src, dst)
            return carry

        lax.fori_loop(0, base_t, tile_body, 0)
        pltpu.sync_copy(dbuf, out_hbm.at[pl.ds(t0 * ETILE, base_t * ETILE)])

        @pl.when(j < rem)
        def _():
            te = b * tpb + wpb * base_t + j
            pltpu.sync_copy(edges_hbm.at[pl.ds(te * 2 * ETILE, 2 * ETILE)],
                            eext)
            for q in range(ETILE // LANES):
                src = eext[pl.ds(q * LANES, LANES)]
                dst = eext[pl.ds(ETILE + q * LANES, LANES)]
                dext[pl.ds(q * LANES, LANES)] = do16(src, dst)
            pltpu.sync_copy(dext, out_hbm.at[pl.ds(te * ETILE, ETILE)])

    return k(coords_flat, ei_lin)


def _lookup_tc(d2, embedding):
    B, _, E = d2.shape
    eblk = 12800
    d_model = embedding.shape[1]

    def body(d2_ref, emb_ref, out_ref):
        d2v = d2_ref[0]
        dist = jnp.sqrt(d2v)
        bkt = jnp.clip((dist * N_BUCKETS).astype(jnp.int32), 0, N_BUCKETS - 1)
        bktb = jnp.broadcast_to(bkt, (N_BUCKETS, eblk))
        iot = lax.broadcasted_iota(jnp.int32, (N_BUCKETS, eblk), 0)
        onehot_t = (bktb == iot).astype(jnp.float32)
        mat = lax.dot_general(
            emb_ref[...], onehot_t,
            dimension_numbers=(((0,), (0,)), ((), ())),
            preferred_element_type=jnp.float32)
        for h in range(N_HEADS):
            out_ref[0, h] = mat[h * D_K:(h + 1) * D_K, :]

    return pl.pallas_call(
        body,
        grid=(B, E // eblk),
        in_specs=[
            pl.BlockSpec((1, 1, eblk), lambda b, i: (b, 0, i)),
            pl.BlockSpec((N_BUCKETS, d_model), lambda b, i: (0, 0)),
        ],
        out_specs=pl.BlockSpec((1, N_HEADS, D_K, eblk),
                               lambda b, i: (b, 0, 0, i)),
        out_shape=jax.ShapeDtypeStruct((B, N_HEADS, D_K, E), jnp.float32),
    )(d2, embedding)


def kernel(coords, edge_index, embedding):
    B, N, _ = coords.shape
    _, E, _ = edge_index.shape
    coords_flat = coords.reshape(-1)
    ntiles = E // ETILE
    ei_lin = (edge_index.astype(jnp.int32)
              .reshape(B, ntiles, ETILE, 2)
              .transpose(0, 1, 3, 2)
              .reshape(-1))
    d2 = _sq_dist_sc(coords_flat, ei_lin, B=B, N=N, E=E)
    out = _lookup_tc(d2.reshape(B, 1, E), embedding)
    return out.transpose(0, 3, 1, 2)


# speedup vs baseline: 139.9721x; 1.0031x over previous
import functools

import jax
import jax.numpy as jnp
from jax import lax
from jax.experimental import pallas as pl
from jax.experimental.pallas import tpu as pltpu
from jax.experimental.pallas import tpu_sc as plsc

N_HEADS = 8
D_K = 16
N_BUCKETS = 32
LANES = 16
NUM_WORKERS = 32
ETILE = 128


def _sq_dist_sc(coords_flat, ei_lin, *, B, N, E, chunk, n_chunks):
    tpb = E // ETILE
    tpc = tpb // n_chunks
    total = B * tpc * ETILE
    wpb = NUM_WORKERS // B
    base_t = tpc // wpb
    rem = tpc - base_t * wpb
    mesh = plsc.VectorSubcoreMesh(core_axis_name="c", subcore_axis_name="s")

    @functools.partial(
        pl.kernel,
        mesh=mesh,
        out_type=jax.ShapeDtypeStruct((total,), jnp.float32),
        scratch_types=[
            pltpu.VMEM((2 * N,), jnp.float32),
            pltpu.VMEM((base_t * 2 * ETILE,), jnp.int32),
            pltpu.VMEM((base_t * ETILE,), jnp.float32),
            pltpu.VMEM((2 * ETILE,), jnp.int32),
            pltpu.VMEM((ETILE,), jnp.float32),
        ],
        compiler_params=pltpu.CompilerParams(needs_layout_passes=False),
    )
    def k(coords_hbm, edges_hbm, out_hbm, coords_v, ebuf, dbuf, eext, dext):
        wid = lax.axis_index("s") * 2 + lax.axis_index("c")
        b = wid // wpb
        j = wid % wpb
        tl0 = j * base_t
        g0 = b * tpb + chunk * tpc + tl0
        pltpu.sync_copy(coords_hbm.at[pl.ds(b * 2 * N, 2 * N)], coords_v)
        pltpu.sync_copy(edges_hbm.at[pl.ds(g0 * 2 * ETILE, base_t * 2 * ETILE)],
                        ebuf)
        def do16(src, dst):
            sx = plsc.load_gather(coords_v, [2 * src])
            sy = plsc.load_gather(coords_v, [2 * src + 1])
            tx = plsc.load_gather(coords_v, [2 * dst])
            ty = plsc.load_gather(coords_v, [2 * dst + 1])
            dx = sx - tx
            dy = sy - ty
            return dx * dx + dy * dy

        def tile_body(t, carry):
            eb = t * 2 * ETILE
            db = t * ETILE
            for q in range(ETILE // LANES):
                src = ebuf[pl.ds(eb + q * LANES, LANES)]
                dst = ebuf[pl.ds(eb + ETILE + q * LANES, LANES)]
                dbuf[pl.ds(db + q * LANES, LANES)] = do16(src, dst)
            return carry

        lax.fori_loop(0, base_t, tile_body, 0)
        pltpu.sync_copy(
            dbuf, out_hbm.at[pl.ds((b * tpc + tl0) * ETILE, base_t * ETILE)])

        @pl.when(j < rem)
        def _():
            tel = wpb * base_t + j
            ge = b * tpb + chunk * tpc + tel
            pltpu.sync_copy(edges_hbm.at[pl.ds(ge * 2 * ETILE, 2 * ETILE)],
                            eext)
            for q in range(ETILE // LANES):
                src = eext[pl.ds(q * LANES, LANES)]
                dst = eext[pl.ds(ETILE + q * LANES, LANES)]
                dext[pl.ds(q * LANES, LANES)] = do16(src, dst)
            pltpu.sync_copy(
                dext, out_hbm.at[pl.ds((b * tpc + tel) * ETILE, ETILE)])

    return k(coords_flat, ei_lin)


def _lookup_tc(d2c, embedding, *, E_full, chunk, n_chunks, prev=None):
    B, _, Ec = d2c.shape
    eblk = 16000
    nblk = Ec // eblk
    d_model = embedding.shape[1]

    def body(*refs):
        d2_ref, emb_ref = refs[0], refs[1]
        out_ref = refs[-1]
        d2v = d2_ref[0]
        dist = jnp.sqrt(d2v)
        bkt = jnp.clip((dist * N_BUCKETS).astype(jnp.int32), 0, N_BUCKETS - 1)
        bktb = jnp.broadcast_to(bkt, (N_BUCKETS, eblk))
        iot = lax.broadcasted_iota(jnp.int32, (N_BUCKETS, eblk), 0)
        onehot_t = (bktb == iot).astype(jnp.float32)
        mat = lax.dot_general(
            emb_ref[...], onehot_t,
            dimension_numbers=(((0,), (0,)), ((), ())),
            preferred_element_type=jnp.float32)
        for h in range(N_HEADS):
            out_ref[0, h] = mat[h * D_K:(h + 1) * D_K, :]

    in_specs = [
        pl.BlockSpec((1, 1, eblk), lambda b, i: (b, 0, i)),
        pl.BlockSpec((N_BUCKETS, d_model), lambda b, i: (0, 0)),
    ]
    args = [d2c, embedding]
    aliases = {}
    if prev is not None:
        in_specs.append(pl.BlockSpec(memory_space=pl.ANY))
        args.append(prev)
        aliases = {2: 0}

    return pl.pallas_call(
        body,
        grid=(B, nblk),
        in_specs=in_specs,
        out_specs=pl.BlockSpec((1, N_HEADS, D_K, eblk),
                               lambda b, i: (b, 0, 0, chunk * nblk + i)),
        out_shape=jax.ShapeDtypeStruct((B, N_HEADS, D_K, E_full), jnp.float32),
        input_output_aliases=aliases,
    )(*args)


def kernel(coords, edge_index, embedding):
    B, N, _ = coords.shape
    _, E, _ = edge_index.shape
    ntiles = E // ETILE
    ei_lin = (edge_index.astype(jnp.int32)
              .reshape(B, ntiles, ETILE, 2)
              .transpose(0, 1, 3, 2)
              .reshape(-1))
    coords_flat = coords.reshape(-1)
    n_chunks = 2
    ec = E // n_chunks
    out = None
    d2s = [_sq_dist_sc(coords_flat, ei_lin, B=B, N=N, E=E,
                       chunk=c, n_chunks=n_chunks)
           for c in range(n_chunks)]
    for c in range(n_chunks):
        out = _lookup_tc(d2s[c].reshape(B, 1, ec), embedding, E_full=E,
                         chunk=c, n_chunks=n_chunks, prev=out)
    return out.transpose(0, 3, 1, 2)


# speedup vs baseline: 140.2770x; 1.0022x over previous
import functools

import jax
import jax.numpy as jnp
from jax import lax
from jax.experimental import pallas as pl
from jax.experimental.pallas import tpu as pltpu
from jax.experimental.pallas import tpu_sc as plsc

N_HEADS = 8
D_K = 16
N_BUCKETS = 32
LANES = 16
NUM_WORKERS = 32
ETILE = 128


def _sq_dist_sc(coords_flat, ei_lin, *, B, N, E):
    tpb = E // ETILE
    total = B * E
    wpb = NUM_WORKERS // B
    base_t = tpb // wpb
    rem = tpb - base_t * wpb
    mesh = plsc.VectorSubcoreMesh(core_axis_name="c", subcore_axis_name="s")

    @functools.partial(
        pl.kernel,
        mesh=mesh,
        out_type=jax.ShapeDtypeStruct((total,), jnp.float32),
        scratch_types=[
            pltpu.VMEM((2 * N,), jnp.float32),
            pltpu.VMEM((base_t * 2 * ETILE,), jnp.int32),
            pltpu.VMEM((base_t * ETILE,), jnp.float32),
            pltpu.VMEM((2 * ETILE,), jnp.int32),
            pltpu.VMEM((ETILE,), jnp.float32),
        ],
        compiler_params=pltpu.CompilerParams(needs_layout_passes=False),
    )
    def k(coords_hbm, edges_hbm, out_hbm, coords_v, ebuf, dbuf, eext, dext):
        wid = lax.axis_index("s") * 2 + lax.axis_index("c")
        b = wid // wpb
        j = wid % wpb
        g0 = b * tpb + j * base_t
        pltpu.sync_copy(coords_hbm.at[pl.ds(b * 2 * N, 2 * N)], coords_v)
        pltpu.sync_copy(edges_hbm.at[pl.ds(g0 * 2 * ETILE, base_t * 2 * ETILE)],
                        ebuf)
        def do16(src, dst):
            sx = plsc.load_gather(coords_v, [2 * src])
            sy = plsc.load_gather(coords_v, [2 * src + 1])
            tx = plsc.load_gather(coords_v, [2 * dst])
            ty = plsc.load_gather(coords_v, [2 * dst + 1])
            dx = sx - tx
            dy = sy - ty
            return dx * dx + dy * dy

        def tile_body(t, carry):
            eb = t * 2 * ETILE
            db = t * ETILE
            for q in range(ETILE // LANES):
                src = ebuf[pl.ds(eb + q * LANES, LANES)]
                dst = ebuf[pl.ds(eb + ETILE + q * LANES, LANES)]
                dbuf[pl.ds(db + q * LANES, LANES)] = do16(src, dst)
            return carry

        lax.fori_loop(0, base_t, tile_body, 0)
        pltpu.sync_copy(dbuf, out_hbm.at[pl.ds(g0 * ETILE, base_t * ETILE)])

        @pl.when(j < rem)
        def _():
            ge = b * tpb + wpb * base_t + j
            pltpu.sync_copy(edges_hbm.at[pl.ds(ge * 2 * ETILE, 2 * ETILE)],
                            eext)
            for q in range(ETILE // LANES):
                src = eext[pl.ds(q * LANES, LANES)]
                dst = eext[pl.ds(ETILE + q * LANES, LANES)]
                dext[pl.ds(q * LANES, LANES)] = do16(src, dst)
            pltpu.sync_copy(dext, out_hbm.at[pl.ds(ge * ETILE, ETILE)])

    return k(coords_flat, ei_lin)


def _lookup_tc(d2c, embedding, *, E_full, chunk, n_chunks, prev=None):
    B, _, Ec = d2c.shape
    eblk = 16000
    nblk = Ec // eblk
    d_model = embedding.shape[1]

    def body(*refs):
        d2_ref, emb_ref = refs[0], refs[1]
        out_ref = refs[-1]
        d2v = d2_ref[0]
        dist = jnp.sqrt(d2v)
        bkt = jnp.clip((dist * N_BUCKETS).astype(jnp.int32), 0, N_BUCKETS - 1)
        bktb = jnp.broadcast_to(bkt, (N_BUCKETS, eblk))
        iot = lax.broadcasted_iota(jnp.int32, (N_BUCKETS, eblk), 0)
        onehot_t = (bktb == iot).astype(jnp.float32)
        mat = lax.dot_general(
            emb_ref[...], onehot_t,
            dimension_numbers=(((0,), (0,)), ((), ())),
            preferred_element_type=jnp.float32)
        for h in range(N_HEADS):
            out_ref[0, h] = mat[h * D_K:(h + 1) * D_K, :]

    in_specs = [
        pl.BlockSpec((1, 1, eblk), lambda b, i: (b, 0, i)),
        pl.BlockSpec((N_BUCKETS, d_model), lambda b, i: (0, 0)),
    ]
    args = [d2c, embedding]
    aliases = {}
    if prev is not None:
        in_specs.append(pl.BlockSpec(memory_space=pl.ANY))
        args.append(prev)
        aliases = {2: 0}

    return pl.pallas_call(
        body,
        grid=(B, nblk),
        in_specs=in_specs,
        out_specs=pl.BlockSpec((1, N_HEADS, D_K, eblk),
                               lambda b, i: (b, 0, 0, chunk * nblk + i)),
        out_shape=jax.ShapeDtypeStruct((B, N_HEADS, D_K, E_full), jnp.float32),
        input_output_aliases=aliases,
    )(*args)


def kernel(coords, edge_index, embedding):
    B, N, _ = coords.shape
    _, E, _ = edge_index.shape
    coords_flat = coords.reshape(-1)
    n_chunks = 2
    ec = E // n_chunks
    d2s = []
    for c in range(n_chunks):
        ei_c = (edge_index[:, c * ec:(c + 1) * ec, :].astype(jnp.int32)
                .reshape(B, ec // ETILE, ETILE, 2)
                .transpose(0, 1, 3, 2)
                .reshape(-1))
        d2s.append(_sq_dist_sc(coords_flat, ei_c, B=B, N=N, E=ec))
    out = None
    for c in range(n_chunks):
        out = _lookup_tc(d2s[c].reshape(B, 1, ec), embedding, E_full=E,
                         chunk=c, n_chunks=n_chunks, prev=out)
    return out.transpose(0, 3, 1, 2)
